# baseline (speedup 1.0000x reference)
import functools

import jax
import jax.numpy as jnp
from jax import lax
from jax.experimental import pallas as pl
from jax.experimental.pallas import tpu as pltpu
from jax.experimental.pallas import tpu_sc as plsc

_B = 4096
_V = 8192
_H = 200
_F = 32
_CW = 3
_PW = 10
_OUT_LEN = _H - _CW + 1
_POOL_OUT = _OUT_LEN // _PW
_T = _POOL_OUT * _PW + _CW - 1
_NW = 32
_ROWS = _B // _NW
_OUT_W = _F * _POOL_OUT
_FLUSH = 16
_SB = _ROWS // (2 * _FLUSH)
_CR = _T + _CW - 1


def _sc_body(x_hbm, tab_hbm, out_hbm, xv, conv, oblk, spm,
             sg0, sg1, sg2, sg3, sg4, sg5, sg6, sg7, so0, so1):
    sid = lax.axis_index("s")
    wid = sid * 2 + lax.axis_index("c")
    base = wid * _ROWS
    lane = lax.iota(jnp.int32, 16)
    sidx_e = (2 * lane) * _POOL_OUT
    sidx_o = (2 * lane + 1) * _POOL_OUT
    zero32 = jnp.zeros((32,), jnp.bfloat16)
    sems_g = (sg0, sg1, sg2, sg3, sg4, sg5, sg6, sg7)
    sems_o = (so0, so1)

    vrows = _V // 16
    for l in range(_CW):
        pltpu.sync_copy(
            tab_hbm.at[l, pl.ds(sid * vrows, vrows)],
            spm.at[l, pl.ds(sid * vrows, vrows)])

    @pl.loop(0, _CR)
    def _zero(i):
        for slot in range(8):
            conv[slot, i, :] = zero32

    plsc.subcore_barrier()

    pltpu.sync_copy(x_hbm.at[pl.ds(base, _ROWS)], xv)

    def fire_gather(r, slot):
        for l in range(_CW):
            for c in range(2):
                pltpu.async_copy(
                    spm.at[l].at[xv.at[r, pl.ds(96 * c, 96)]],
                    conv.at[slot, pl.ds(2 - l + 96 * c, 96)],
                    sems_g[slot], add=True)

    def wait_gather(slot):
        for _ in range(3):
            pltpu.make_async_copy(
                spm.at[0, pl.ds(0, 2 * 96)],
                conv.at[slot, pl.ds(0, 2 * 96)], sems_g[slot]).wait()

    def flush_ref(sb, half):
        return out_hbm.at[pl.ds(base + sb * 32 + half * _FLUSH, _FLUSH)]

    for p in range(8):
        fire_gather(p, p)

    @pl.loop(0, _SB)
    def _superblock(sb):
        for half in range(2):
            @pl.when(sb > 0)
            def _():
                pltpu.make_async_copy(
                    oblk.at[half], flush_ref(sb, half), sems_o[half]).wait()

            for k in range(_FLUSH):
                r = sb * 32 + half * _FLUSH + k
                slot = k % 8
                wait_gather(slot)

                @pl.loop(0, _POOL_OUT)
                def _grp(g):
                    j0 = g * _PW + 2
                    acc_a = jnp.zeros((32,), jnp.bfloat16)
                    acc_b = jnp.zeros((32,), jnp.bfloat16)
                    for s in range(0, _PW, 2):
                        ca = conv[slot, j0 + s, :]
                        acc_a = acc_a + jnp.maximum(ca, 0)
                        conv[slot, j0 + s, :] = zero32
                        cb = conv[slot, j0 + s + 1, :]
                        acc_b = acc_b + jnp.maximum(cb, 0)
                        conv[slot, j0 + s + 1, :] = zero32
                    krow = jnp.full((16,), k, jnp.int32)
                    a_e, a_o = plsc.unpack(
                        acc_a + acc_b, format=plsc.PackFormat.INTERLEAVED)
                    plsc.store_scatter(
                        oblk.at[half], [krow, sidx_e + g], a_e)
                    plsc.store_scatter(
                        oblk.at[half], [krow, sidx_o + g], a_o)

                @pl.when(r + 8 < _ROWS)
                def _():
                    fire_gather(r + 8, slot)

            pltpu.async_copy(oblk.at[half], flush_ref(sb, half), sems_o[half])

    for half in range(2):
        pltpu.make_async_copy(
            oblk.at[half], flush_ref(_SB - 1, half), sems_o[half]).wait()


_sc_kernel = functools.partial(
    pl.kernel,
    out_type=jax.ShapeDtypeStruct((_B, _OUT_W), jnp.float32),
    mesh=plsc.VectorSubcoreMesh(core_axis_name="c", subcore_axis_name="s"),
    scratch_types=[
        pltpu.VMEM((_ROWS, _H), jnp.int32),
        pltpu.VMEM((8, _CR, _F), jnp.bfloat16),
        pltpu.VMEM((2, _FLUSH, _OUT_W), jnp.float32),
        pltpu.VMEM_SHARED((_CW, _V, _F), jnp.bfloat16),
        pltpu.SemaphoreType.DMA,
        pltpu.SemaphoreType.DMA,
        pltpu.SemaphoreType.DMA,
        pltpu.SemaphoreType.DMA,
        pltpu.SemaphoreType.DMA,
        pltpu.SemaphoreType.DMA,
        pltpu.SemaphoreType.DMA,
        pltpu.SemaphoreType.DMA,
        pltpu.SemaphoreType.DMA,
        pltpu.SemaphoreType.DMA,
    ],
    compiler_params=pltpu.CompilerParams(
        use_tc_tiling_on_sc=False, needs_layout_passes=False),
)(_sc_body)


@jax.jit
def kernel(x, lut0, lut1, lut2):
    tab = jnp.stack([lut0, lut1, lut2]).astype(jnp.bfloat16)
    return _sc_kernel(x, tab)
